# baseline (speedup 1.0000x reference)
import jax
import jax.numpy as jnp
from jax.experimental import pallas as pl

EPS = 1e-5


def _ln_body(x_ref, pe_ref, g_ref, b_ref, o_ref):
    x = x_ref[:, 0, 0, :]
    pe = pe_ref[...]
    h = x + pe
    mean = jnp.mean(h, axis=-1, keepdims=True)
    d = h - mean
    var = jnp.mean(d * d, axis=-1, keepdims=True)
    o_ref[:, 0, 0, :] = d * jax.lax.rsqrt(var + EPS) * g_ref[...] + b_ref[...]


@jax.jit
def kernel(x, pos_table, gamma, beta):
    S, B, D = x.shape
    S_BLK = 64
    grid = (S // S_BLK, B)
    g2 = gamma.reshape(1, D)
    b2 = beta.reshape(1, D)
    x4 = x.reshape(S, B, 1, D)
    out = pl.pallas_call(
        _ln_body,
        grid=grid,
        in_specs=[
            pl.BlockSpec((S_BLK, 1, 1, D), lambda i, j: (i, j, 0, 0)),
            pl.BlockSpec((S_BLK, D), lambda i, j: (i, 0)),
            pl.BlockSpec((1, D), lambda i, j: (0, 0)),
            pl.BlockSpec((1, D), lambda i, j: (0, 0)),
        ],
        out_specs=pl.BlockSpec((S_BLK, 1, 1, D), lambda i, j: (i, j, 0, 0)),
        out_shape=jax.ShapeDtypeStruct((S, B, 1, D), x.dtype),
    )(x4, pos_table[:S], g2, b2)
    return out.reshape(S, B, D)
